# baseline (speedup 1.0000x reference)
import jax
import jax.numpy as jnp
from jax.experimental import pallas as pl

NEG = -100000000.0
EPS = 1.1920929e-07
BIGI = 2**30


def _heads_kernel(vml_ref, vmm_ref, pmp_ref, pmm_ref,
                  selvm_ref, selpm_ref, lp_ref, ent_ref):
    pid = pl.program_id(0)
    br = vml_ref.shape[0]
    vml = vml_ref[...]
    vmm = vmm_ref[...]
    x = jnp.where(vmm, NEG, vml)
    m = jnp.max(x, axis=1, keepdims=True)
    e = jnp.exp(x - m)
    em = jnp.where(vmm, 0.0, e)
    s = jnp.sum(e, axis=1, keepdims=True)
    su = jnp.sum(em, axis=1, keepdims=True)
    sxe = jnp.sum(em * x, axis=1, keepdims=True)
    logs = jnp.log(s)
    lse = m + logs
    vm_lp = -logs[:, 0]
    vm_ent = (lse[:, 0] * su[:, 0] - sxe[:, 0]) / s[:, 0]
    ii = jax.lax.broadcasted_iota(jnp.int32, x.shape, 1)
    sel_vm = jnp.min(jnp.where(x == m, ii, BIGI), axis=1)

    pp = pmp_ref[...]
    pmm = pmm_ref[...]
    p = jnp.where(pmm, 0.0, pp)
    S = jnp.sum(p, axis=1, keepdims=True)
    un = jnp.where(pmm, 0.0, 1.0)
    cnt = jnp.sum(un, axis=1, keepdims=True)
    small = S < 0.0001
    p2 = jnp.where(small, un, p)
    S2 = jnp.where(small, cnt, S)
    q = p2 / S2
    lq = jnp.log(jnp.clip(q, EPS, 1.0 - EPS))
    pm_ent = -jnp.sum(lq * q, axis=1)
    mq = jnp.max(q, axis=1, keepdims=True)
    jj = jax.lax.broadcasted_iota(jnp.int32, q.shape, 1)
    sel_pm = jnp.min(jnp.where(q == mq, jj, BIGI), axis=1)
    pm_lp = jnp.log(jnp.clip(mq[:, 0], EPS, 1.0 - EPS))

    selvm_ref[0, 0, :] = sel_vm
    selpm_ref[0, 0, :] = sel_pm
    lp_ref[0, 0, :] = vm_lp + pm_lp
    ent_ref[0, 0, :] = vm_ent + pm_ent


def kernel(vm_logits, vm_mask, pm_probs, pm_mask):
    B = vm_logits.shape[0]
    NV = vm_logits.shape[1]
    NP = pm_probs.shape[1]
    G = 8
    BR = B // G
    out_spec = pl.BlockSpec((1, 1, BR), lambda i: (i, 0, 0))
    out = pl.pallas_call(
        _heads_kernel,
        grid=(G,),
        in_specs=[
            pl.BlockSpec((BR, NV), lambda i: (i, 0)),
            pl.BlockSpec((BR, NV), lambda i: (i, 0)),
            pl.BlockSpec((BR, NP), lambda i: (i, 0)),
            pl.BlockSpec((BR, NP), lambda i: (i, 0)),
        ],
        out_specs=(out_spec, out_spec, out_spec, out_spec),
        out_shape=(
            jax.ShapeDtypeStruct((G, 1, BR), jnp.int32),
            jax.ShapeDtypeStruct((G, 1, BR), jnp.int32),
            jax.ShapeDtypeStruct((G, 1, BR), jnp.float32),
            jax.ShapeDtypeStruct((G, 1, BR), jnp.float32),
        ),
    )(vm_logits, vm_mask, pm_probs, pm_mask)
    return tuple(o.reshape(B) for o in out)


# speedup vs baseline: 1.0823x; 1.0823x over previous
import jax
import jax.numpy as jnp
from jax.experimental import pallas as pl

NEG = -100000000.0
EPS = 1.1920929e-07
BIGI = 2**30


def _heads_kernel(vml_ref, vmm_ref, pmp_ref, pmm_ref,
                  selvm_ref, selpm_ref, lp_ref, ent_ref):
    pid = pl.program_id(0)
    br = vml_ref.shape[0]
    vml = vml_ref[...]
    vmm = vmm_ref[...]
    x = jnp.where(vmm, NEG, vml)
    m = jnp.max(x, axis=1, keepdims=True)
    e = jnp.exp(x - m)
    em = jnp.where(vmm, 0.0, e)
    s = jnp.sum(e, axis=1, keepdims=True)
    su = jnp.sum(em, axis=1, keepdims=True)
    sxe = jnp.sum(em * x, axis=1, keepdims=True)
    logs = jnp.log(s)
    lse = m + logs
    vm_lp = -logs[:, 0]
    vm_ent = (lse[:, 0] * su[:, 0] - sxe[:, 0]) / s[:, 0]
    ii = jax.lax.broadcasted_iota(jnp.int32, x.shape, 1)
    sel_vm = jnp.min(jnp.where(x == m, ii, BIGI), axis=1)

    pp = pmp_ref[...]
    pmm = pmm_ref[...]
    p = jnp.where(pmm, 0.0, pp)
    S = jnp.sum(p, axis=1, keepdims=True)
    un = jnp.where(pmm, 0.0, 1.0)
    cnt = jnp.sum(un, axis=1, keepdims=True)
    small = S < 0.0001
    p2 = jnp.where(small, un, p)
    S2 = jnp.where(small, cnt, S)
    q = p2 / S2
    lq = jnp.log(jnp.clip(q, EPS, 1.0 - EPS))
    pm_ent = -jnp.sum(lq * q, axis=1)
    mq = jnp.max(q, axis=1, keepdims=True)
    jj = jax.lax.broadcasted_iota(jnp.int32, q.shape, 1)
    sel_pm = jnp.min(jnp.where(q == mq, jj, BIGI), axis=1)
    pm_lp = jnp.log(jnp.clip(mq[:, 0], EPS, 1.0 - EPS))

    selvm_ref[0, 0, :] = sel_vm
    selpm_ref[0, 0, :] = sel_pm
    lp_ref[0, 0, :] = vm_lp + pm_lp
    ent_ref[0, 0, :] = vm_ent + pm_ent


def kernel(vm_logits, vm_mask, pm_probs, pm_mask):
    B = vm_logits.shape[0]
    NV = vm_logits.shape[1]
    NP = pm_probs.shape[1]
    G = 4
    BR = B // G
    out_spec = pl.BlockSpec((1, 1, BR), lambda i: (i, 0, 0))
    out = pl.pallas_call(
        _heads_kernel,
        grid=(G,),
        in_specs=[
            pl.BlockSpec((BR, NV), lambda i: (i, 0)),
            pl.BlockSpec((BR, NV), lambda i: (i, 0)),
            pl.BlockSpec((BR, NP), lambda i: (i, 0)),
            pl.BlockSpec((BR, NP), lambda i: (i, 0)),
        ],
        out_specs=(out_spec, out_spec, out_spec, out_spec),
        out_shape=(
            jax.ShapeDtypeStruct((G, 1, BR), jnp.int32),
            jax.ShapeDtypeStruct((G, 1, BR), jnp.int32),
            jax.ShapeDtypeStruct((G, 1, BR), jnp.float32),
            jax.ShapeDtypeStruct((G, 1, BR), jnp.float32),
        ),
    )(vm_logits, vm_mask, pm_probs, pm_mask)
    return tuple(o.reshape(B) for o in out)


# speedup vs baseline: 1.6129x; 1.4903x over previous
import jax
import jax.numpy as jnp
from jax.experimental import pallas as pl

NEG = -100000000.0
EPS = 1.1920929e-07
BIGI = 2**30


def _heads_kernel(vml_ref, vmm_ref, pmp_ref, pmm_ref,
                  selvm_ref, selpm_ref, lp_ref, ent_ref):
    pid = pl.program_id(0)
    br = vml_ref.shape[0]
    vml = vml_ref[...]
    vmm = vmm_ref[...]
    x = jnp.where(vmm, NEG, vml)
    m = jnp.max(x, axis=1, keepdims=True)
    e = jnp.exp(x - m)
    em = jnp.where(vmm, 0.0, e)
    s = jnp.sum(e, axis=1, keepdims=True)
    su = jnp.sum(em, axis=1, keepdims=True)
    sxe = jnp.sum(em * x, axis=1, keepdims=True)
    logs = jnp.log(s)
    lse = m + logs
    vm_lp = -logs[:, 0]
    vm_ent = (lse[:, 0] * su[:, 0] - sxe[:, 0]) / s[:, 0]
    ii = jax.lax.broadcasted_iota(jnp.int32, x.shape, 1)
    sel_vm = jnp.min(jnp.where(x == m, ii, BIGI), axis=1)

    pp = pmp_ref[...]
    pmm = pmm_ref[...]
    p = jnp.where(pmm, 0.0, pp)
    S = jnp.sum(p, axis=1, keepdims=True)
    un = jnp.where(pmm, 0.0, 1.0)
    cnt = jnp.sum(un, axis=1, keepdims=True)
    small = S < 0.0001
    p2 = jnp.where(small, un, p)
    S2 = jnp.where(small, cnt, S)
    q = p2 / S2
    lq = jnp.log(jnp.clip(q, EPS, 1.0 - EPS))
    pm_ent = -jnp.sum(lq * q, axis=1)
    mq = jnp.max(q, axis=1, keepdims=True)
    jj = jax.lax.broadcasted_iota(jnp.int32, q.shape, 1)
    sel_pm = jnp.min(jnp.where(q == mq, jj, BIGI), axis=1)
    pm_lp = jnp.log(jnp.clip(mq[:, 0], EPS, 1.0 - EPS))

    selvm_ref[0, 0, :] = sel_vm
    selpm_ref[0, 0, :] = sel_pm
    lp_ref[0, 0, :] = vm_lp + pm_lp
    ent_ref[0, 0, :] = vm_ent + pm_ent


def kernel(vm_logits, vm_mask, pm_probs, pm_mask):
    B = vm_logits.shape[0]
    NV = vm_logits.shape[1]
    NP = pm_probs.shape[1]
    G = 1
    BR = B // G
    out_spec = pl.BlockSpec((1, 1, BR), lambda i: (i, 0, 0))
    out = pl.pallas_call(
        _heads_kernel,
        grid=(G,),
        in_specs=[
            pl.BlockSpec((BR, NV), lambda i: (i, 0)),
            pl.BlockSpec((BR, NV), lambda i: (i, 0)),
            pl.BlockSpec((BR, NP), lambda i: (i, 0)),
            pl.BlockSpec((BR, NP), lambda i: (i, 0)),
        ],
        out_specs=(out_spec, out_spec, out_spec, out_spec),
        out_shape=(
            jax.ShapeDtypeStruct((G, 1, BR), jnp.int32),
            jax.ShapeDtypeStruct((G, 1, BR), jnp.int32),
            jax.ShapeDtypeStruct((G, 1, BR), jnp.float32),
            jax.ShapeDtypeStruct((G, 1, BR), jnp.float32),
        ),
    )(vm_logits, vm_mask, pm_probs, pm_mask)
    return tuple(o.reshape(B) for o in out)
